# baseline (speedup 1.0000x reference)
import functools

import jax
import jax.numpy as jnp
from jax import lax
from jax.experimental import pallas as pl
from jax.experimental.pallas import tpu as pltpu
from jax.experimental.pallas import tpu_sc as plsc

_B = 4


def _sc_gather_bcast(idx2d, table):
    n_sel, pad = idx2d.shape
    row = table.shape[1]
    info = plsc.get_sparse_core_info()
    nw = info.num_cores * info.num_subcores
    assert n_sel % nw == 0
    per_w = n_sel // nw

    mesh = plsc.VectorSubcoreMesh(core_axis_name="c", subcore_axis_name="s")

    @functools.partial(
        pl.kernel,
        mesh=mesh,
        out_type=jax.ShapeDtypeStruct((_B * n_sel, row), jnp.float32),
        scratch_types=[
            pltpu.VMEM((pad,), jnp.int32),
            pltpu.VMEM((per_w, row), jnp.float32),
            pltpu.SemaphoreType.DMA,
            pltpu.SemaphoreType.DMA,
        ],
    )
    def body(idx_hbm, table_hbm, out_hbm, idx_v, rows_v, gsem, wsem):
        wid = lax.axis_index("s") * info.num_cores + lax.axis_index("c")
        base = wid * per_w
        pltpu.sync_copy(idx_hbm.at[base], idx_v)
        pltpu.async_copy(table_hbm.at[idx_v.at[pl.ds(0, per_w)]], rows_v, gsem).wait()
        copies = [
            pltpu.async_copy(rows_v, out_hbm.at[pl.ds(b * n_sel + base, per_w)], wsem)
            for b in range(_B)
        ]
        for c in copies:
            c.wait()

    return body(idx2d, table)


def kernel(indices, batch_size, prompts):
    del batch_size
    n_pool, length, dim = prompts.shape
    n_sel = indices.shape[0]
    table = prompts.reshape(n_pool, length * dim)
    idx2d = jnp.broadcast_to(indices.astype(jnp.int32)[:, None], (n_sel, 8))
    flat = _sc_gather_bcast(idx2d, table)
    return flat.reshape(_B, n_sel * length, 1, dim)


# speedup vs baseline: 2.0936x; 2.0936x over previous
import functools

import jax
import jax.numpy as jnp
from jax import lax
from jax.experimental import pallas as pl
from jax.experimental.pallas import tpu as pltpu
from jax.experimental.pallas import tpu_sc as plsc

_B = 4


def _sc_gather_bcast(idx2d, table):
    n_sel, pad = idx2d.shape
    n_pool, length, dim = table.shape
    info = plsc.get_sparse_core_info()
    nw = info.num_cores * info.num_subcores
    assert n_sel % nw == 0
    per_w = n_sel // nw

    mesh = plsc.VectorSubcoreMesh(core_axis_name="c", subcore_axis_name="s")

    @functools.partial(
        pl.kernel,
        mesh=mesh,
        out_type=jax.ShapeDtypeStruct((_B * n_sel, length, dim), jnp.float32),
        scratch_types=[
            pltpu.VMEM((pad,), jnp.int32),
            pltpu.VMEM((per_w, length, dim), jnp.float32),
            pltpu.SemaphoreType.DMA,
            pltpu.SemaphoreType.DMA,
        ],
    )
    def body(idx_hbm, table_hbm, out_hbm, idx_v, rows_v, gsem, wsem):
        wid = lax.axis_index("s") * info.num_cores + lax.axis_index("c")
        base = wid * per_w
        pltpu.sync_copy(idx_hbm.at[base], idx_v)
        pltpu.async_copy(table_hbm.at[idx_v.at[pl.ds(0, per_w)]], rows_v, gsem).wait()
        copies = [
            pltpu.async_copy(rows_v, out_hbm.at[pl.ds(b * n_sel + base, per_w)], wsem)
            for b in range(_B)
        ]
        for c in copies:
            c.wait()

    return body(idx2d, table)


def kernel(indices, batch_size, prompts):
    del batch_size
    n_pool, length, dim = prompts.shape
    n_sel = indices.shape[0]
    idx2d = jnp.broadcast_to(indices.astype(jnp.int32)[:, None], (n_sel, 8))
    flat = _sc_gather_bcast(idx2d, prompts)
    return flat.reshape(_B, n_sel * length, 1, dim)


# speedup vs baseline: 2.8449x; 1.3589x over previous
import functools

import jax
import jax.numpy as jnp
from jax import lax
from jax.experimental import pallas as pl
from jax.experimental.pallas import tpu as pltpu
from jax.experimental.pallas import tpu_sc as plsc

_B = 4


def _sc_gather_bcast(idx2d, table):
    n_sel, pad = idx2d.shape
    n_pool, length, dim = table.shape
    info = plsc.get_sparse_core_info()
    nw = info.num_cores * info.num_subcores
    assert n_sel % nw == 0
    per_w = n_sel // nw

    mesh = plsc.VectorSubcoreMesh(core_axis_name="c", subcore_axis_name="s")

    @functools.partial(
        pl.kernel,
        mesh=mesh,
        out_type=jax.ShapeDtypeStruct((_B, n_sel * length, 1, dim), jnp.float32),
        scratch_types=[
            pltpu.VMEM((pad,), jnp.int32),
            pltpu.VMEM((per_w, length, dim), jnp.float32),
            pltpu.SemaphoreType.DMA,
            pltpu.SemaphoreType.DMA,
        ],
    )
    def body(idx_hbm, table_hbm, out_hbm, idx_v, rows_v, gsem, wsem):
        wid = lax.axis_index("s") * info.num_cores + lax.axis_index("c")
        base = wid * per_w
        pltpu.sync_copy(idx_hbm.at[base], idx_v)
        pltpu.async_copy(table_hbm.at[idx_v.at[pl.ds(0, per_w)]], rows_v, gsem).wait()
        copies = [
            pltpu.async_copy(
                rows_v.at[0],
                out_hbm.at[b, pl.ds(base * length, per_w * length), 0],
                wsem,
            )
            for b in range(_B)
        ]
        for c in copies:
            c.wait()

    return body(idx2d, table)


def kernel(indices, batch_size, prompts):
    del batch_size
    n_pool, length, dim = prompts.shape
    n_sel = indices.shape[0]
    idx2d = jnp.broadcast_to(indices.astype(jnp.int32)[:, None], (n_sel, 8))
    return _sc_gather_bcast(idx2d, prompts)


# speedup vs baseline: 2.8550x; 1.0035x over previous
import functools

import jax
import jax.numpy as jnp
from jax import lax
from jax.experimental import pallas as pl
from jax.experimental.pallas import tpu as pltpu
from jax.experimental.pallas import tpu_sc as plsc

_B = 4


def _sc_gather_bcast(idx, table):
    n_sel = idx.shape[0]
    n_pool, length, dim = table.shape
    info = plsc.get_sparse_core_info()
    lanes = info.num_lanes
    nw = info.num_cores * info.num_subcores
    assert n_sel % nw == 0
    per_w = n_sel // nw

    mesh = plsc.VectorSubcoreMesh(core_axis_name="c", subcore_axis_name="s")

    @functools.partial(
        pl.kernel,
        mesh=mesh,
        out_type=jax.ShapeDtypeStruct((_B, n_sel * length, 1, dim), jnp.float32),
        scratch_types=[
            pltpu.VMEM((n_sel + lanes,), jnp.int32),
            pltpu.VMEM((lanes,), jnp.int32),
            pltpu.VMEM((per_w, length, dim), jnp.float32),
            pltpu.SemaphoreType.DMA,
            pltpu.SemaphoreType.DMA,
        ],
    )
    def body(idx_hbm, table_hbm, out_hbm, idx_all, idx_mine, rows_v, gsem, wsem):
        wid = lax.axis_index("s") * info.num_cores + lax.axis_index("c")
        base = wid * per_w
        pltpu.sync_copy(idx_hbm, idx_all.at[pl.ds(0, n_sel)])
        window = idx_all[pl.ds(base, lanes)]
        idx_mine[...] = jnp.full((lanes,), window[0], jnp.int32)
        pltpu.async_copy(table_hbm.at[idx_mine.at[pl.ds(0, per_w)]], rows_v, gsem).wait()

        def write(b, carry):
            pltpu.sync_copy(
                rows_v.at[0],
                out_hbm.at[b, pl.ds(base * length, per_w * length), 0],
            )
            return carry

        lax.fori_loop(0, _B, write, 0)

    return body(idx, table)


def kernel(indices, batch_size, prompts):
    del batch_size
    return _sc_gather_bcast(indices.astype(jnp.int32), prompts)
